# baseline (speedup 1.0000x reference)
import jax
import jax.numpy as jnp
from jax import lax
from jax.experimental import pallas as pl
from jax.experimental.pallas import tpu as pltpu
from jax.experimental.pallas import tpu_sc as plsc

D = 256
HID = 256
OUT = 64

NW = 32
CHUNK = 56
SPLIT_CHUNKS = (5, 5, 4)
N_PARTS = (8960, 8960, 7168)
N_OUT = 25000
TC_BLOCK = 1792


def _make_sc_gather(chunks_per_w):
  rows_per_w = CHUNK * chunks_per_w

  def _sc_gather(imr_hbm, gr_hbm, idx_hbm, x_hbm,
                 idx_v, b1a, b1b, b2a, b2b, sem_g, sem_wa, sem_wb):
    wid = lax.axis_index("s") * 2 + lax.axis_index("c")
    base = wid * rows_per_w
    bufs1 = (b1a, b1b)
    bufs2 = (b2a, b2b)
    sem_w = (sem_wa, sem_wb)
    pltpu.sync_copy(idx_hbm.at[pl.ds(base, rows_per_w)], idx_v)
    idx0 = idx_v.at[pl.ds(0, CHUNK)]
    gcur = [pltpu.async_copy(imr_hbm.at[idx0], bufs1[0], sem_g),
            pltpu.async_copy(gr_hbm.at[idx0], bufs2[0], sem_g)]
    pending = []
    for c in range(chunks_per_w):
        cur = c % 2
        nxt = (c + 1) % 2
        for d in gcur:
            d.wait()
        row0 = base + c * CHUNK
        pending.append([
            pltpu.async_copy(
                bufs1[cur], x_hbm.at[pl.ds(row0, CHUNK), pl.ds(0, D)],
                sem_w[cur]),
            pltpu.async_copy(
                bufs2[cur], x_hbm.at[pl.ds(row0, CHUNK), pl.ds(D, D)],
                sem_w[cur]),
        ])
        if c + 1 < chunks_per_w:
            if len(pending) >= 2:
                for d in pending.pop(0):
                    d.wait()
            idx_c = idx_v.at[pl.ds((c + 1) * CHUNK, CHUNK)]
            gcur = [pltpu.async_copy(imr_hbm.at[idx_c], bufs1[nxt], sem_g),
                    pltpu.async_copy(gr_hbm.at[idx_c], bufs2[nxt], sem_g)]
    for grp in pending:
        for d in grp:
            d.wait()

  return _sc_gather


def _gather_rows(imr, gr, idx_part, chunks_per_w):
    mesh = plsc.VectorSubcoreMesh(core_axis_name="c", subcore_axis_name="s")
    f = pl.kernel(
        _make_sc_gather(chunks_per_w),
        out_type=jax.ShapeDtypeStruct(
            (NW * CHUNK * chunks_per_w, 2 * D), jnp.float32),
        mesh=mesh,
        scratch_types=[
            pltpu.VMEM((CHUNK * chunks_per_w,), jnp.int32),
            pltpu.VMEM((CHUNK, D), jnp.float32),
            pltpu.VMEM((CHUNK, D), jnp.float32),
            pltpu.VMEM((CHUNK, D), jnp.float32),
            pltpu.VMEM((CHUNK, D), jnp.float32),
            pltpu.SemaphoreType.DMA,
            pltpu.SemaphoreType.DMA,
            pltpu.SemaphoreType.DMA,
        ],
    )
    return f(imr, gr, idx_part)


def _mlp_body(*refs):
    x_ref, w1_ref, w2_ref, b1_ref, b2_ref = refs[:5]
    o_ref = refs[-1]
    h = jnp.dot(x_ref[...], w1_ref[...], preferred_element_type=jnp.float32)
    h = jnp.maximum(h + b1_ref[...], 0.0)
    o_t = jax.lax.dot_general(
        w2_ref[...], h, (((0,), (1,)), ((), ())),
        preferred_element_type=jnp.float32)
    o_ref[...] = o_t + b2_ref[...]


def _mlp(x, w1, w2, b1r, b2r, o_prev, half):
    off = sum(N_PARTS[:half]) // TC_BLOCK
    in_specs = [
        pl.BlockSpec((TC_BLOCK, 2 * D), lambda i: (i, 0)),
        pl.BlockSpec((2 * D, HID), lambda i: (0, 0)),
        pl.BlockSpec((HID, OUT), lambda i: (0, 0)),
        pl.BlockSpec((1, HID), lambda i: (0, 0)),
        pl.BlockSpec((OUT, 1), lambda i: (0, 0)),
    ]
    args = [x, w1, w2, b1r, b2r]
    aliases = {}
    if half:
        in_specs.append(pl.BlockSpec((OUT, TC_BLOCK), lambda i: (0, i + off)))
        args.append(o_prev)
        aliases = {5: 0}
    return pl.pallas_call(
        _mlp_body,
        grid=(N_PARTS[half] // TC_BLOCK,),
        in_specs=in_specs,
        out_specs=pl.BlockSpec((OUT, TC_BLOCK), lambda i: (0, i + off)),
        out_shape=jax.ShapeDtypeStruct((OUT, N_OUT), jnp.float32),
        input_output_aliases=aliases,
    )(*args)


def kernel(input_molecule_representations, graph_representations,
           graphs_requiring_node_choices, W1, b1, W2, b2):
    n_sel = graphs_requiring_node_choices.shape[0]
    idx = graphs_requiring_node_choices.astype(jnp.int32)
    idx_pad = jnp.concatenate(
        [idx, jnp.zeros((sum(N_PARTS) - n_sel,), jnp.int32)])
    b1r = b1.reshape(1, HID)
    b2r = b2.reshape(OUT, 1)
    out = None
    row = 0
    for h in range(len(N_PARTS)):
        x = _gather_rows(
            input_molecule_representations, graph_representations,
            idx_pad[row:row + N_PARTS[h]], SPLIT_CHUNKS[h])
        out = _mlp(x, W1, W2, b1r, b2r, out, h)
        row += N_PARTS[h]
    return out.T
